# Initial kernel scaffold; baseline (speedup 1.0000x reference)
import jax
import jax.numpy as jnp
from jax.experimental import pallas as pl


def kernel(features, w1, w2):
    raise NotImplementedError("write your pallas kernel here")


# speedup vs baseline: 15.1674x; 15.1674x over previous
import functools

import jax
import jax.numpy as jnp
from jax import lax
from jax.experimental import pallas as pl

KP1 = 31
BISECT_ITERS = 26


def _prep_body(f_ref, w1_ref, w2_ref, emb_ref):
    h = f_ref[...] * w1_ref[...]
    h = jnp.maximum(h, 0.0) * w2_ref[...]
    n2 = jnp.sum(h * h, axis=1, keepdims=True)
    norm = jnp.sqrt(n2)
    emb_ref[...] = h / jnp.maximum(norm, 1e-12)


def _main_body(a_ref, b_ref, out_ref, *, bm):
    a = a_ref[...]
    b = b_ref[...]
    sim = lax.dot_general(a, b, (((1,), (1,)), ((), ())),
                          preferred_element_type=jnp.float32)

    lo = jnp.full((bm, 1), -1.01, jnp.float32)
    hi = jnp.full((bm, 1), 1.01, jnp.float32)

    def body(_, carry):
        lo, hi = carry
        mid = 0.5 * (lo + hi)
        cnt = jnp.sum((sim >= mid).astype(jnp.float32), axis=1, keepdims=True)
        ge = cnt >= KP1
        return jnp.where(ge, mid, lo), jnp.where(ge, hi, mid)

    lo, hi = lax.fori_loop(0, BISECT_ITERS, body, (lo, hi))
    out_ref[...] = jnp.where((sim >= lo) & (sim > 0.0), sim, 0.0)


def kernel(features, w1, w2):
    n, d = features.shape
    w1r = w1.reshape(1, d)
    w2r = w2.reshape(1, d)

    emb = pl.pallas_call(
        _prep_body,
        out_shape=jax.ShapeDtypeStruct((n, d), jnp.float32),
    )(features, w1r, w2r)

    bm = 200 if n % 200 == 0 else n
    grid = n // bm

    out = pl.pallas_call(
        functools.partial(_main_body, bm=bm),
        grid=(grid,),
        in_specs=[
            pl.BlockSpec((bm, d), lambda i: (i, 0)),
            pl.BlockSpec((n, d), lambda i: (0, 0)),
        ],
        out_specs=pl.BlockSpec((bm, n), lambda i: (i, 0)),
        out_shape=jax.ShapeDtypeStruct((n, n), jnp.float32),
    )(emb, emb)
    return out


# speedup vs baseline: 17.5353x; 1.1561x over previous
import functools

import jax
import jax.numpy as jnp
from jax import lax
from jax.experimental import pallas as pl

KP1 = 31
BISECT_ITERS = 22


def _prep_body(f_ref, w1_ref, w2_ref, emb_ref):
    h = f_ref[...] * w1_ref[...]
    h = jnp.maximum(h, 0.0) * w2_ref[...]
    n2 = jnp.sum(h * h, axis=1, keepdims=True)
    norm = jnp.sqrt(n2)
    emb_ref[...] = h / jnp.maximum(norm, 1e-12)


def _main_body(a_ref, b_ref, out_ref, *, bm):
    a = a_ref[...]
    b = b_ref[...]
    sim = lax.dot_general(a, b, (((1,), (1,)), ((), ())),
                          preferred_element_type=jnp.float32)

    lo = jnp.full((bm, 1), -0.01, jnp.float32)
    hi = jnp.full((bm, 1), 1.01, jnp.float32)

    def body(_, carry):
        lo, hi = carry
        mid = 0.5 * (lo + hi)
        cnt = jnp.sum((sim >= mid).astype(jnp.float32), axis=1, keepdims=True)
        ge = cnt >= KP1
        return jnp.where(ge, mid, lo), jnp.where(ge, hi, mid)

    lo, hi = lax.fori_loop(0, BISECT_ITERS, body, (lo, hi))
    out_ref[...] = jnp.where((sim >= lo) & (sim > 0.0), sim, 0.0)


def kernel(features, w1, w2):
    n, d = features.shape
    w1r = w1.reshape(1, d)
    w2r = w2.reshape(1, d)

    emb = pl.pallas_call(
        _prep_body,
        out_shape=jax.ShapeDtypeStruct((n, d), jnp.float32),
    )(features, w1r, w2r)

    bm = 200 if n % 200 == 0 else n
    grid = n // bm

    out = pl.pallas_call(
        functools.partial(_main_body, bm=bm),
        grid=(grid,),
        in_specs=[
            pl.BlockSpec((bm, d), lambda i: (i, 0)),
            pl.BlockSpec((n, d), lambda i: (0, 0)),
        ],
        out_specs=pl.BlockSpec((bm, n), lambda i: (i, 0)),
        out_shape=jax.ShapeDtypeStruct((n, n), jnp.float32),
    )(emb, emb)
    return out
